# baseline (speedup 1.0000x reference)
import functools

import jax
import jax.numpy as jnp
from jax import lax
from jax.experimental import pallas as pl
from jax.experimental.pallas import tpu as pltpu
from jax.experimental.pallas import tpu_sc as plsc

K = 1024
DIM = 256
MOMENTUM = 0.1
LANES = 16
NUM_CORES = 2
NUM_SUBCORES = 16
NUM_WORKERS = NUM_CORES * NUM_SUBCORES
ROWS_PER_W = K // NUM_WORKERS
NVEC = DIM // LANES
ROW_UNROLL = 4

_GATHER_DNUMS = lax.GatherDimensionNumbers(
    offset_dims=(), collapsed_slice_dims=(0,), start_index_map=(0,))


def _shuffle(v, idx):
    return lax.gather(v, idx.reshape(LANES, 1), _GATHER_DNUMS, (1,),
                      mode=lax.GatherScatterMode.PROMISE_IN_BOUNDS)


def _lane_sum(v):
    lanes = lax.iota(jnp.int32, LANES)
    for k in (8, 4, 2, 1):
        v = v + _shuffle(v, lanes ^ k)
    return v


def _rsqrt_nr(s):
    bits = lax.bitcast_convert_type(s, jnp.int32)
    y = lax.bitcast_convert_type(jnp.int32(0x5F3759DF) - (bits >> 1), jnp.float32)
    for _ in range(3):
        y = y * (1.5 - 0.5 * s * y * y)
    return y


@functools.cache
def _build_update():
    mesh = plsc.VectorSubcoreMesh(
        core_axis_name="c", subcore_axis_name="s",
        num_cores=NUM_CORES, num_subcores=NUM_SUBCORES)

    @functools.partial(
        pl.kernel,
        out_type=jax.ShapeDtypeStruct((K, DIM), jnp.float32),
        mesh=mesh,
        scratch_types=[
            pltpu.VMEM((LANES,), jnp.int32),
            pltpu.VMEM((ROWS_PER_W, DIM), jnp.float32),
            pltpu.VMEM((ROWS_PER_W, DIM), jnp.float32),
            pltpu.VMEM((ROWS_PER_W, DIM), jnp.float32),
            pltpu.SemaphoreType.DMA,
            pltpu.SemaphoreType.DMA,
        ],
    )
    def update(pt_hbm, cent_hbm, a0_hbm, a1_hbm, a2_hbm, out_hbm,
               pt_v, att_v, cent_v, out_v, sem_a, sem_c):
        wid = lax.axis_index("s") * NUM_CORES + lax.axis_index("c")
        base = wid * ROWS_PER_W
        rows = pl.ds(base, ROWS_PER_W)

        cent_dma = pltpu.async_copy(cent_hbm.at[rows], cent_v, sem_c)
        pltpu.sync_copy(pt_hbm, pt_v)
        pt = pt_v[...][0]

        @pl.when(pt == 0)
        def _():
            pltpu.async_copy(a0_hbm.at[rows], att_v, sem_a).wait()

        @pl.when(pt == 1)
        def _():
            pltpu.async_copy(a1_hbm.at[rows], att_v, sem_a).wait()

        @pl.when(pt == 2)
        def _():
            pltpu.async_copy(a2_hbm.at[rows], att_v, sem_a).wait()

        cent_dma.wait()

        def row_group(g, carry):
            r0 = g * ROW_UNROLL
            for i in range(ROW_UNROLL):
                r = r0 + i
                u = []
                acc = None
                for j in range(NVEC):
                    a = att_v[r, pl.ds(j * LANES, LANES)]
                    c = cent_v[r, pl.ds(j * LANES, LANES)]
                    v = (1.0 - MOMENTUM) * a + MOMENTUM * c
                    u.append(v)
                    acc = v * v if acc is None else acc + v * v
                inv = _rsqrt_nr(_lane_sum(acc))
                for j in range(NVEC):
                    out_v[r, pl.ds(j * LANES, LANES)] = u[j] * inv
            return carry

        lax.fori_loop(0, ROWS_PER_W // ROW_UNROLL, row_group, 0)
        pltpu.sync_copy(out_v, out_hbm.at[rows])

    return update


def kernel(new_centroids, cluster_counts, position_type, attractors_0,
           attractors_1, attractors_2):
    del cluster_counts
    update = _build_update()
    pt = jnp.broadcast_to(jnp.asarray(position_type, jnp.int32), (LANES,))
    return update(pt, new_centroids, attractors_0, attractors_1, attractors_2)
